# Initial kernel scaffold; baseline (speedup 1.0000x reference)
import jax
import jax.numpy as jnp
from jax.experimental import pallas as pl


def kernel(idx, token_embed_table):
    raise NotImplementedError("write your pallas kernel here")


# speedup vs baseline: 1.0367x; 1.0367x over previous
import functools

import jax
import jax.numpy as jnp
from jax import lax
from jax.experimental import pallas as pl
from jax.experimental.pallas import tpu as pltpu
from jax.experimental.pallas import tpu_sc as plsc

_NC, _NS = 2, 16
_NW = _NC * _NS
_CHUNK = 40


@functools.lru_cache(maxsize=None)
def _build(n_rows, vocab, d):
    assert n_rows % _NW == 0
    per_w = n_rows // _NW
    assert per_w % (2 * _CHUNK) == 0
    n_pairs = per_w // (2 * _CHUNK)
    mesh = plsc.VectorSubcoreMesh(core_axis_name="c", subcore_axis_name="s")

    @functools.partial(
        pl.kernel,
        mesh=mesh,
        out_type=jax.ShapeDtypeStruct((n_rows, d), jnp.float32),
        scratch_types=[
            pltpu.VMEM((per_w,), jnp.int32),
            pltpu.VMEM((_CHUNK, d), jnp.float32),
            pltpu.VMEM((_CHUNK, d), jnp.float32),
            pltpu.SemaphoreType.DMA,
            pltpu.SemaphoreType.DMA,
            pltpu.SemaphoreType.DMA,
            pltpu.SemaphoreType.DMA,
        ],
        compiler_params=pltpu.CompilerParams(use_tc_tiling_on_sc=False),
    )
    def gather_kernel(idx_hbm, table_hbm, out_hbm,
                      idx_v, buf0, buf1, gs0, gs1, ss0, ss1):
        wid = lax.axis_index("s") * _NC + lax.axis_index("c")
        base = wid * per_w
        pltpu.sync_copy(idx_hbm.at[pl.ds(base, per_w)], idx_v)

        def gather(c, buf, sem):
            return pltpu.make_async_copy(
                table_hbm.at[idx_v.at[pl.ds(c * _CHUNK, _CHUNK)]], buf, sem)

        def scatter(c, buf, sem):
            return pltpu.make_async_copy(
                buf, out_hbm.at[pl.ds(base + c * _CHUNK, _CHUNK)], sem)

        gather(0, buf0, gs0).start()

        def body(j, carry):
            c0 = 2 * j
            c1 = c0 + 1

            @pl.when(j > 0)
            def _():
                scatter(c0 - 1, buf1, ss1).wait()
            gather(c1, buf1, gs1).start()
            gather(c0, buf0, gs0).wait()
            scatter(c0, buf0, ss0).start()

            @pl.when(j < n_pairs - 1)
            def _():
                scatter(c0, buf0, ss0).wait()
                gather(c0 + 2, buf0, gs0).start()
            gather(c1, buf1, gs1).wait()
            scatter(c1, buf1, ss1).start()
            return carry

        lax.fori_loop(0, n_pairs, body, 0)
        scatter(0, buf0, ss0).wait()
        scatter(0, buf1, ss1).wait()

    return gather_kernel


def kernel(idx, token_embed_table):
    b, t = idx.shape
    vocab, d = token_embed_table.shape
    flat_idx = idx.reshape(b * t).astype(jnp.int32)
    out = _build(b * t, vocab, d)(flat_idx, token_embed_table)
    return out.reshape(b, t, d)
